# baseline (speedup 1.0000x reference)
import functools

import jax
import jax.numpy as jnp
from jax import lax
from jax.experimental import pallas as pl
from jax.experimental.pallas import tpu as pltpu
from jax.experimental.pallas import tpu_sc as plsc

_INTERPRET = False

LANES = 128
NEG = -1e30
BLK = 512
BLK_SHIFT = BLK.bit_length() - 1
SC_TILES = 32
SL = 16


def _dot(a, b):
    return jnp.dot(a.astype(jnp.bfloat16), b.astype(jnp.bfloat16),
                   preferred_element_type=jnp.float32)


def _router_body(x_ref, w1_ref, b1_ref, w2_ref, b2_ref, w3_ref, b3_ref,
                 p_ref, aux_ref, ph_ref):
    xb = x_ref[...]
    h1 = jax.nn.relu(_dot(xb, w1_ref[...]) + b1_ref[...])
    h2 = jax.nn.relu(_dot(h1, w2_ref[...]) + b2_ref[...])
    logits = _dot(h2, w3_ref[...]) + b3_ref[...]

    m = jnp.max(logits, axis=1, keepdims=True)
    ex = jnp.exp(logits - m)
    s = jnp.sum(ex, axis=1, keepdims=True)
    p = ex / s
    p_ref[...] = p

    idx = jax.lax.broadcasted_iota(jnp.int32, p.shape, 1)
    m1 = jnp.max(p, axis=1, keepdims=True)
    e0 = jnp.min(jnp.where(p == m1, idx, LANES), axis=1, keepdims=True)
    pm = jnp.where(idx == e0, -1.0, p)
    m2 = jnp.max(pm, axis=1, keepdims=True)
    e1 = jnp.min(jnp.where(pm == m2, idx, LANES), axis=1, keepdims=True)

    e0f = e0.astype(jnp.float32)
    e1f = e1.astype(jnp.float32)
    aux = jnp.where(idx == 0, e0f,
          jnp.where(idx == 1, e1f,
          jnp.where(idx == 2, m1,
          jnp.where(idx == 3, m2, 0.0))))
    aux_ref[...] = aux

    ind = (jnp.where(idx == e0, 1.0, 0.0) + jnp.where(idx == e1, 1.0, 0.0))
    tb = ind.shape[0]
    nrow = tb // 128
    ph_ref[...] = jnp.sum(ind.reshape(1, nrow, 128, LANES), axis=2)


def _run_router(x, Wr1, br1, Wr2, br2, Wr3, br3):
    B, D = x.shape
    HID = Wr1.shape[1]
    H2 = Wr2.shape[1]
    E = Wr3.shape[1]
    TB = min(512, B)
    nb = B // TB
    nrow = TB // 128

    w3p = jnp.zeros((H2, LANES), Wr3.dtype).at[:, :E].set(Wr3)
    b3p = jnp.full((1, LANES), NEG, br3.dtype).at[0, :E].set(br3)

    out_shapes = (
        jax.ShapeDtypeStruct((B, LANES), jnp.float32),
        jax.ShapeDtypeStruct((B, LANES), jnp.float32),
        jax.ShapeDtypeStruct((nb, nrow, LANES), jnp.float32),
    )
    return pl.pallas_call(
        _router_body,
        grid=(nb,),
        in_specs=[
            pl.BlockSpec((TB, D), lambda i: (i, 0)),
            pl.BlockSpec((D, HID), lambda i: (0, 0)),
            pl.BlockSpec((1, HID), lambda i: (0, 0)),
            pl.BlockSpec((HID, H2), lambda i: (0, 0)),
            pl.BlockSpec((1, H2), lambda i: (0, 0)),
            pl.BlockSpec((H2, LANES), lambda i: (0, 0)),
            pl.BlockSpec((1, LANES), lambda i: (0, 0)),
        ],
        out_specs=(
            pl.BlockSpec((TB, LANES), lambda i: (i, 0)),
            pl.BlockSpec((TB, LANES), lambda i: (i, 0)),
            pl.BlockSpec((1, nrow, LANES), lambda i: (i, 0, 0)),
        ),
        out_shape=out_shapes,
        interpret=_INTERPRET,
    )(x, Wr1, br1.reshape(1, HID), Wr2, br2.reshape(1, H2), w3p, b3p)


def _gather16(v, idx):
    dn = lax.GatherDimensionNumbers(offset_dims=(), collapsed_slice_dims=(0,),
                                    start_index_map=(0,))
    return lax.gather(v, idx[:, None], dn, slice_sizes=(1,),
                      mode=lax.GatherScatterMode.PROMISE_IN_BOUNDS)


def _make_dispatch(B, D, E, npad, nblk_pad):
    tpb = B // SC_TILES
    nv = tpb // SL
    rows_c = 32
    nchunk = 2 * tpb // rows_c
    mesh = plsc.VectorSubcoreMesh(core_axis_name="c", subcore_axis_name="s")
    info = plsc.get_sparse_core_info()
    nc = info.num_cores

    @functools.partial(
        pl.kernel, mesh=mesh,
        out_type=(
            jax.ShapeDtypeStruct((npad, D), jnp.float32),
            jax.ShapeDtypeStruct((2 * B,), jnp.int32),
            jax.ShapeDtypeStruct((nblk_pad,), jnp.int32),
        ),
        scratch_types=[
            pltpu.VMEM((tpb,), jnp.int32),
            pltpu.VMEM((tpb,), jnp.int32),
            pltpu.VMEM((SC_TILES, LANES), jnp.float32),
            pltpu.VMEM((2 * tpb,), jnp.int32),
            pltpu.VMEM((nchunk, rows_c), jnp.int32),
            pltpu.VMEM((nblk_pad,), jnp.int32),
            pltpu.VMEM((rows_c, D), jnp.float32),
            pltpu.VMEM((rows_c, D), jnp.float32),
            pltpu.SemaphoreType.DMA,
            pltpu.SemaphoreType.DMA,
        ],
    )
    def dispatch(e0_hbm, e1_hbm, ph_hbm, x_hbm,
                 xg_hbm, pos_hbm, bexp_hbm,
                 e0_v, e1_v, ph_v, posbuf, pos3, bexp_v,
                 rb0, rb1, sem0, sem1):
        wid = lax.axis_index("s") * nc + lax.axis_index("c")
        base = wid * tpb

        pltpu.sync_copy(e0_hbm.at[pl.ds(base, tpb)], e0_v)
        pltpu.sync_copy(e1_hbm.at[pl.ds(base, tpb)], e1_v)
        pltpu.sync_copy(ph_hbm, ph_v)

        lane = jnp.arange(SL, dtype=jnp.int32)
        total = jnp.zeros((SL,), jnp.int32)
        before = jnp.zeros((SL,), jnp.int32)
        for w in range(SC_TILES):
            v = ph_v[w, pl.ds(0, SL)].astype(jnp.int32)
            total = total + v
            before = before + jnp.where(w < wid, v, 0)

        padded = (total + (BLK - 1)) & (-BLK)
        s = padded
        for k in (1, 2, 4, 8):
            sh = _gather16(s, jnp.maximum(lane - k, 0))
            s = s + jnp.where(lane >= k, sh, 0)
        basev = s - padded
        ctr = basev + before

        for half, eref in ((0, e0_v), (1, e1_v)):
            for i in range(nv):
                veid = eref[pl.ds(i * SL, SL)]
                g = _gather16(ctr, veid)
                rank = jnp.zeros((SL,), jnp.int32)
                for k in range(1, SL):
                    sh = _gather16(veid, jnp.maximum(lane - k, 0))
                    hit = (sh == veid) & (lane >= k)
                    rank = rank + jnp.where(hit, 1, 0)
                upd = jnp.zeros((SL,), jnp.int32)
                for j in range(SL):
                    vj = _gather16(veid, jnp.full((SL,), j, jnp.int32))
                    upd = upd + jnp.where(lane == vj, 1, 0)
                ctr = ctr + upd
                pos = g + rank
                o = half * tpb + i * SL
                posbuf[pl.ds(o, SL)] = pos
                pos3[o // rows_c, pl.ds(o % rows_c, SL)] = pos

        pltpu.sync_copy(posbuf.at[pl.ds(0, tpb)],
                        pos_hbm.at[pl.ds(base, tpb)])
        pltpu.sync_copy(posbuf.at[pl.ds(tpb, tpb)],
                        pos_hbm.at[pl.ds(B + base, tpb)])

        bufs = (rb0, rb1)
        sems = (sem0, sem1)
        pending = [None, None]
        cpt = tpb // rows_c
        for c in range(nchunk):
            b = c % 2
            if pending[b] is not None:
                pending[b].wait()
            tok0 = base + (c % cpt) * rows_c
            pltpu.sync_copy(x_hbm.at[pl.ds(tok0, rows_c)], bufs[b])
            cp = pltpu.make_async_copy(bufs[b], xg_hbm.at[pos3.at[c]],
                                       sems[b])
            cp.start()
            pending[b] = cp
        for b in range(2):
            if pending[b] is not None:
                pending[b].wait()

        @pl.when(wid == 0)
        def _():
            bstarts = jnp.right_shift(basev, BLK_SHIFT)
            for vi in range(nblk_pad // SL):
                jv = jnp.arange(SL, dtype=jnp.int32) + vi * SL
                acc = jnp.full((SL,), -1, jnp.int32)
                for e in range(E):
                    bse = _gather16(bstarts, jnp.full((SL,), e, jnp.int32))
                    acc = acc + jnp.where(jv >= bse, 1, 0)
                bexp_v[pl.ds(vi * SL, SL)] = acc
            pltpu.sync_copy(bexp_v, bexp_hbm)

    return dispatch


def _grouped_body(bexp_ref, xg_ref, w1_ref, b1_ref, w2_ref, b2_ref, y_ref):
    xb = xg_ref[...]
    h = jax.nn.relu(_dot(xb, w1_ref[0]) + b1_ref[0])
    y_ref[...] = _dot(h, w2_ref[0]) + b2_ref[0]


def _run_grouped(bexp, xg, We1, be1, We2, be2, nblk):
    E, D, DFF = We1.shape
    C = We2.shape[2]
    npad = xg.shape[0]

    grid_spec = pltpu.PrefetchScalarGridSpec(
        num_scalar_prefetch=1,
        grid=(nblk,),
        in_specs=[
            pl.BlockSpec((BLK, D), lambda b, be: (b, 0)),
            pl.BlockSpec((1, D, DFF), lambda b, be: (be[b], 0, 0)),
            pl.BlockSpec((1, 1, DFF), lambda b, be: (be[b], 0, 0)),
            pl.BlockSpec((1, DFF, C), lambda b, be: (be[b], 0, 0)),
            pl.BlockSpec((1, 1, C), lambda b, be: (be[b], 0, 0)),
        ],
        out_specs=pl.BlockSpec((BLK, C), lambda b, be: (b, 0)),
    )
    return pl.pallas_call(
        _grouped_body,
        grid_spec=grid_spec,
        out_shape=jax.ShapeDtypeStruct((npad, C), jnp.float32),
        interpret=_INTERPRET,
    )(bexp, xg, We1, be1.reshape(E, 1, DFF), We2, be2.reshape(E, 1, C))


def _make_gather(B, C, npad):
    tpb = B // SC_TILES
    rows_c = 32
    cpt = tpb // rows_c
    mesh = plsc.VectorSubcoreMesh(core_axis_name="c", subcore_axis_name="s")
    info = plsc.get_sparse_core_info()
    nc = info.num_cores

    @functools.partial(
        pl.kernel, mesh=mesh,
        out_type=(
            jax.ShapeDtypeStruct((B, C), jnp.float32),
            jax.ShapeDtypeStruct((B, C), jnp.float32),
        ),
        scratch_types=[
            pltpu.VMEM((tpb,), jnp.int32),
            pltpu.VMEM((tpb,), jnp.int32),
            pltpu.VMEM((rows_c, C), jnp.float32),
            pltpu.VMEM((rows_c, C), jnp.float32),
            pltpu.SemaphoreType.DMA,
            pltpu.SemaphoreType.DMA,
            pltpu.SemaphoreType.DMA,
        ],
    )
    def gather(y_hbm, pos_hbm, y0_hbm, y1_hbm,
               p0_v, p1_v, rb0, rb1, gsem, wsem0, wsem1):
        wid = lax.axis_index("s") * nc + lax.axis_index("c")
        base = wid * tpb

        pltpu.sync_copy(pos_hbm.at[pl.ds(base, tpb)], p0_v)
        pltpu.sync_copy(pos_hbm.at[pl.ds(B + base, tpb)], p1_v)

        bufs = (rb0, rb1)
        wsems = (wsem0, wsem1)
        pending = [None, None]
        for half, pv, yout in ((0, p0_v, y0_hbm), (1, p1_v, y1_hbm)):
            for c in range(cpt):
                b = (half * cpt + c) % 2
                if pending[b] is not None:
                    pending[b].wait()
                gcp = pltpu.make_async_copy(
                    y_hbm.at[pv.at[pl.ds(c * rows_c, rows_c)]],
                    bufs[b], gsem)
                gcp.start()
                gcp.wait()
                wp = pltpu.make_async_copy(
                    bufs[b], yout.at[pl.ds(base + c * rows_c, rows_c)],
                    wsems[b])
                wp.start()
                pending[b] = wp
        for b in range(2):
            if pending[b] is not None:
                pending[b].wait()

    return gather


def _combine_body(y0_ref, y1_ref, aux_ref, out_ref):
    w0 = aux_ref[:, 2:3]
    w1 = aux_ref[:, 3:4]
    out_ref[...] = 0.5 * (w0 * y0_ref[...] + w1 * y1_ref[...])


def _run_combine(y0, y1, aux):
    B, C = y0.shape
    TB = min(512, B)
    return pl.pallas_call(
        _combine_body,
        grid=(B // TB,),
        in_specs=[
            pl.BlockSpec((TB, C), lambda i: (i, 0)),
            pl.BlockSpec((TB, C), lambda i: (i, 0)),
            pl.BlockSpec((TB, LANES), lambda i: (i, 0)),
        ],
        out_specs=pl.BlockSpec((TB, C), lambda i: (i, 0)),
        out_shape=jax.ShapeDtypeStruct((B, C), jnp.float32),
        interpret=_INTERPRET,
    )(y0, y1, aux)


def kernel(x, Wr1, br1, Wr2, br2, Wr3, br3, We1, be1, We2, be2):
    B, D = x.shape
    E, _, DFF = We1.shape
    C = We2.shape[2]
    nblk = 2 * B // BLK + E
    nblk_pad = ((nblk + SL - 1) // SL) * SL
    npad = nblk * BLK

    x_bf = x.astype(jnp.bfloat16)
    router_p_pad, aux, phist3 = _run_router(x_bf, Wr1, br1, Wr2, br2, Wr3, br3)
    router_p = router_p_pad[:, :E]

    e0 = aux[:, 0].astype(jnp.int32)
    e1 = aux[:, 1].astype(jnp.int32)
    phist = phist3.reshape(SC_TILES, LANES)

    dispatch = _make_dispatch(B, D, E, npad, nblk_pad)
    xg, pos, bexp = dispatch(e0, e1, phist, x)

    y = _run_grouped(bexp, xg, We1.astype(jnp.bfloat16), be1,
                     We2.astype(jnp.bfloat16), be2, nblk)

    gather = _make_gather(B, C, npad)
    y0, y1 = gather(y, pos)

    out = _run_combine(y0, y1, aux)

    lb_loss = jnp.asarray(0.0, jnp.float32)
    return (out, router_p, lb_loss)


# speedup vs baseline: 1.0025x; 1.0025x over previous
import functools

import jax
import jax.numpy as jnp
from jax import lax
from jax.experimental import pallas as pl
from jax.experimental.pallas import tpu as pltpu
from jax.experimental.pallas import tpu_sc as plsc

_INTERPRET = False

LANES = 128
NEG = -1e30
BLK = 256
BLK_SHIFT = BLK.bit_length() - 1
SC_TILES = 32
SL = 16


def _dot(a, b):
    return jnp.dot(a.astype(jnp.bfloat16), b.astype(jnp.bfloat16),
                   preferred_element_type=jnp.float32)


def _router_body(x_ref, w1_ref, b1_ref, w2_ref, b2_ref, w3_ref, b3_ref,
                 p_ref, aux_ref, ph_ref):
    xb = x_ref[...]
    h1 = jax.nn.relu(_dot(xb, w1_ref[...]) + b1_ref[...])
    h2 = jax.nn.relu(_dot(h1, w2_ref[...]) + b2_ref[...])
    logits = _dot(h2, w3_ref[...]) + b3_ref[...]

    m = jnp.max(logits, axis=1, keepdims=True)
    ex = jnp.exp(logits - m)
    s = jnp.sum(ex, axis=1, keepdims=True)
    p = ex / s
    p_ref[...] = p

    idx = jax.lax.broadcasted_iota(jnp.int32, p.shape, 1)
    m1 = jnp.max(p, axis=1, keepdims=True)
    e0 = jnp.min(jnp.where(p == m1, idx, LANES), axis=1, keepdims=True)
    pm = jnp.where(idx == e0, -1.0, p)
    m2 = jnp.max(pm, axis=1, keepdims=True)
    e1 = jnp.min(jnp.where(pm == m2, idx, LANES), axis=1, keepdims=True)

    e0f = e0.astype(jnp.float32)
    e1f = e1.astype(jnp.float32)
    aux = jnp.where(idx == 0, e0f,
          jnp.where(idx == 1, e1f,
          jnp.where(idx == 2, m1,
          jnp.where(idx == 3, m2, 0.0))))
    aux_ref[...] = aux

    ind = (jnp.where(idx == e0, 1.0, 0.0) + jnp.where(idx == e1, 1.0, 0.0))
    tb = ind.shape[0]
    nrow = tb // 128
    ph_ref[...] = jnp.sum(ind.reshape(1, nrow, 128, LANES), axis=2)


def _run_router(x, Wr1, br1, Wr2, br2, Wr3, br3):
    B, D = x.shape
    HID = Wr1.shape[1]
    H2 = Wr2.shape[1]
    E = Wr3.shape[1]
    TB = min(512, B)
    nb = B // TB
    nrow = TB // 128

    w3p = jnp.zeros((H2, LANES), Wr3.dtype).at[:, :E].set(Wr3)
    b3p = jnp.full((1, LANES), NEG, br3.dtype).at[0, :E].set(br3)

    out_shapes = (
        jax.ShapeDtypeStruct((B, LANES), jnp.float32),
        jax.ShapeDtypeStruct((B, LANES), jnp.float32),
        jax.ShapeDtypeStruct((nb, nrow, LANES), jnp.float32),
    )
    return pl.pallas_call(
        _router_body,
        grid=(nb,),
        in_specs=[
            pl.BlockSpec((TB, D), lambda i: (i, 0)),
            pl.BlockSpec((D, HID), lambda i: (0, 0)),
            pl.BlockSpec((1, HID), lambda i: (0, 0)),
            pl.BlockSpec((HID, H2), lambda i: (0, 0)),
            pl.BlockSpec((1, H2), lambda i: (0, 0)),
            pl.BlockSpec((H2, LANES), lambda i: (0, 0)),
            pl.BlockSpec((1, LANES), lambda i: (0, 0)),
        ],
        out_specs=(
            pl.BlockSpec((TB, LANES), lambda i: (i, 0)),
            pl.BlockSpec((TB, LANES), lambda i: (i, 0)),
            pl.BlockSpec((1, nrow, LANES), lambda i: (i, 0, 0)),
        ),
        out_shape=out_shapes,
        interpret=_INTERPRET,
    )(x, Wr1, br1.reshape(1, HID), Wr2, br2.reshape(1, H2), w3p, b3p)


def _gather16(v, idx):
    dn = lax.GatherDimensionNumbers(offset_dims=(), collapsed_slice_dims=(0,),
                                    start_index_map=(0,))
    return lax.gather(v, idx[:, None], dn, slice_sizes=(1,),
                      mode=lax.GatherScatterMode.PROMISE_IN_BOUNDS)


def _make_dispatch(B, D, E, npad, nblk_pad):
    tpb = B // SC_TILES
    nv = tpb // SL
    rows_c = 32
    nchunk = 2 * tpb // rows_c
    mesh = plsc.VectorSubcoreMesh(core_axis_name="c", subcore_axis_name="s")
    info = plsc.get_sparse_core_info()
    nc = info.num_cores

    @functools.partial(
        pl.kernel, mesh=mesh,
        out_type=(
            jax.ShapeDtypeStruct((npad, D), jnp.float32),
            jax.ShapeDtypeStruct((2 * B,), jnp.int32),
            jax.ShapeDtypeStruct((nblk_pad,), jnp.int32),
        ),
        scratch_types=[
            pltpu.VMEM((tpb,), jnp.int32),
            pltpu.VMEM((tpb,), jnp.int32),
            pltpu.VMEM((SC_TILES, LANES), jnp.float32),
            pltpu.VMEM((2 * tpb,), jnp.int32),
            pltpu.VMEM((nchunk, rows_c), jnp.int32),
            pltpu.VMEM((nblk_pad,), jnp.int32),
            pltpu.VMEM((rows_c, D), jnp.float32),
            pltpu.VMEM((rows_c, D), jnp.float32),
            pltpu.SemaphoreType.DMA,
            pltpu.SemaphoreType.DMA,
        ],
    )
    def dispatch(e0_hbm, e1_hbm, ph_hbm, x_hbm,
                 xg_hbm, pos_hbm, bexp_hbm,
                 e0_v, e1_v, ph_v, posbuf, pos3, bexp_v,
                 rb0, rb1, sem0, sem1):
        wid = lax.axis_index("s") * nc + lax.axis_index("c")
        base = wid * tpb

        pltpu.sync_copy(e0_hbm.at[pl.ds(base, tpb)], e0_v)
        pltpu.sync_copy(e1_hbm.at[pl.ds(base, tpb)], e1_v)
        pltpu.sync_copy(ph_hbm, ph_v)

        lane = jnp.arange(SL, dtype=jnp.int32)
        total = jnp.zeros((SL,), jnp.int32)
        before = jnp.zeros((SL,), jnp.int32)
        for w in range(SC_TILES):
            v = ph_v[w, pl.ds(0, SL)].astype(jnp.int32)
            total = total + v
            before = before + jnp.where(w < wid, v, 0)

        padded = (total + (BLK - 1)) & (-BLK)
        s = padded
        for k in (1, 2, 4, 8):
            sh = _gather16(s, jnp.maximum(lane - k, 0))
            s = s + jnp.where(lane >= k, sh, 0)
        basev = s - padded
        ctr = basev + before

        for half, eref in ((0, e0_v), (1, e1_v)):
            for i in range(nv):
                veid = eref[pl.ds(i * SL, SL)]
                g = _gather16(ctr, veid)
                rank = jnp.zeros((SL,), jnp.int32)
                for k in range(1, SL):
                    sh = _gather16(veid, jnp.maximum(lane - k, 0))
                    hit = (sh == veid) & (lane >= k)
                    rank = rank + jnp.where(hit, 1, 0)
                upd = jnp.zeros((SL,), jnp.int32)
                for j in range(SL):
                    vj = _gather16(veid, jnp.full((SL,), j, jnp.int32))
                    upd = upd + jnp.where(lane == vj, 1, 0)
                ctr = ctr + upd
                pos = g + rank
                o = half * tpb + i * SL
                posbuf[pl.ds(o, SL)] = pos
                pos3[o // rows_c, pl.ds(o % rows_c, SL)] = pos

        pltpu.sync_copy(posbuf.at[pl.ds(0, tpb)],
                        pos_hbm.at[pl.ds(base, tpb)])
        pltpu.sync_copy(posbuf.at[pl.ds(tpb, tpb)],
                        pos_hbm.at[pl.ds(B + base, tpb)])

        bufs = (rb0, rb1)
        sems = (sem0, sem1)
        pending = [None, None]
        cpt = tpb // rows_c
        for c in range(nchunk):
            b = c % 2
            if pending[b] is not None:
                pending[b].wait()
            tok0 = base + (c % cpt) * rows_c
            pltpu.sync_copy(x_hbm.at[pl.ds(tok0, rows_c)], bufs[b])
            cp = pltpu.make_async_copy(bufs[b], xg_hbm.at[pos3.at[c]],
                                       sems[b])
            cp.start()
            pending[b] = cp
        for b in range(2):
            if pending[b] is not None:
                pending[b].wait()

        @pl.when(wid == 0)
        def _():
            bstarts = jnp.right_shift(basev, BLK_SHIFT)
            for vi in range(nblk_pad // SL):
                jv = jnp.arange(SL, dtype=jnp.int32) + vi * SL
                acc = jnp.full((SL,), -1, jnp.int32)
                for e in range(E):
                    bse = _gather16(bstarts, jnp.full((SL,), e, jnp.int32))
                    acc = acc + jnp.where(jv >= bse, 1, 0)
                bexp_v[pl.ds(vi * SL, SL)] = acc
            pltpu.sync_copy(bexp_v, bexp_hbm)

    return dispatch


def _grouped_body(bexp_ref, xg_ref, w1_ref, b1_ref, w2_ref, b2_ref, y_ref):
    xb = xg_ref[...]
    h = jax.nn.relu(_dot(xb, w1_ref[0]) + b1_ref[0])
    y_ref[...] = _dot(h, w2_ref[0]) + b2_ref[0]


def _run_grouped(bexp, xg, We1, be1, We2, be2, nblk):
    E, D, DFF = We1.shape
    C = We2.shape[2]
    npad = xg.shape[0]

    grid_spec = pltpu.PrefetchScalarGridSpec(
        num_scalar_prefetch=1,
        grid=(nblk,),
        in_specs=[
            pl.BlockSpec((BLK, D), lambda b, be: (b, 0)),
            pl.BlockSpec((1, D, DFF), lambda b, be: (be[b], 0, 0)),
            pl.BlockSpec((1, 1, DFF), lambda b, be: (be[b], 0, 0)),
            pl.BlockSpec((1, DFF, C), lambda b, be: (be[b], 0, 0)),
            pl.BlockSpec((1, 1, C), lambda b, be: (be[b], 0, 0)),
        ],
        out_specs=pl.BlockSpec((BLK, C), lambda b, be: (b, 0)),
    )
    return pl.pallas_call(
        _grouped_body,
        grid_spec=grid_spec,
        out_shape=jax.ShapeDtypeStruct((npad, C), jnp.float32),
        interpret=_INTERPRET,
    )(bexp, xg, We1, be1.reshape(E, 1, DFF), We2, be2.reshape(E, 1, C))


def _make_gather(B, C, npad):
    tpb = B // SC_TILES
    rows_c = 32
    cpt = tpb // rows_c
    mesh = plsc.VectorSubcoreMesh(core_axis_name="c", subcore_axis_name="s")
    info = plsc.get_sparse_core_info()
    nc = info.num_cores

    @functools.partial(
        pl.kernel, mesh=mesh,
        out_type=(
            jax.ShapeDtypeStruct((B, C), jnp.float32),
            jax.ShapeDtypeStruct((B, C), jnp.float32),
        ),
        scratch_types=[
            pltpu.VMEM((tpb,), jnp.int32),
            pltpu.VMEM((tpb,), jnp.int32),
            pltpu.VMEM((rows_c, C), jnp.float32),
            pltpu.VMEM((rows_c, C), jnp.float32),
            pltpu.SemaphoreType.DMA,
            pltpu.SemaphoreType.DMA,
            pltpu.SemaphoreType.DMA,
        ],
    )
    def gather(y_hbm, pos_hbm, y0_hbm, y1_hbm,
               p0_v, p1_v, rb0, rb1, gsem, wsem0, wsem1):
        wid = lax.axis_index("s") * nc + lax.axis_index("c")
        base = wid * tpb

        pltpu.sync_copy(pos_hbm.at[pl.ds(base, tpb)], p0_v)
        pltpu.sync_copy(pos_hbm.at[pl.ds(B + base, tpb)], p1_v)

        bufs = (rb0, rb1)
        wsems = (wsem0, wsem1)
        pending = [None, None]
        for half, pv, yout in ((0, p0_v, y0_hbm), (1, p1_v, y1_hbm)):
            for c in range(cpt):
                b = (half * cpt + c) % 2
                if pending[b] is not None:
                    pending[b].wait()
                gcp = pltpu.make_async_copy(
                    y_hbm.at[pv.at[pl.ds(c * rows_c, rows_c)]],
                    bufs[b], gsem)
                gcp.start()
                gcp.wait()
                wp = pltpu.make_async_copy(
                    bufs[b], yout.at[pl.ds(base + c * rows_c, rows_c)],
                    wsems[b])
                wp.start()
                pending[b] = wp
        for b in range(2):
            if pending[b] is not None:
                pending[b].wait()

    return gather


def _combine_body(y0_ref, y1_ref, aux_ref, out_ref):
    w0 = aux_ref[:, 2:3]
    w1 = aux_ref[:, 3:4]
    out_ref[...] = 0.5 * (w0 * y0_ref[...] + w1 * y1_ref[...])


def _run_combine(y0, y1, aux):
    B, C = y0.shape
    TB = min(512, B)
    return pl.pallas_call(
        _combine_body,
        grid=(B // TB,),
        in_specs=[
            pl.BlockSpec((TB, C), lambda i: (i, 0)),
            pl.BlockSpec((TB, C), lambda i: (i, 0)),
            pl.BlockSpec((TB, LANES), lambda i: (i, 0)),
        ],
        out_specs=pl.BlockSpec((TB, C), lambda i: (i, 0)),
        out_shape=jax.ShapeDtypeStruct((B, C), jnp.float32),
        interpret=_INTERPRET,
    )(y0, y1, aux)


def kernel(x, Wr1, br1, Wr2, br2, Wr3, br3, We1, be1, We2, be2):
    B, D = x.shape
    E, _, DFF = We1.shape
    C = We2.shape[2]
    nblk = 2 * B // BLK + E
    nblk_pad = ((nblk + SL - 1) // SL) * SL
    npad = nblk * BLK

    x_bf = x.astype(jnp.bfloat16)
    router_p_pad, aux, phist3 = _run_router(x_bf, Wr1, br1, Wr2, br2, Wr3, br3)
    router_p = router_p_pad[:, :E]

    e0 = aux[:, 0].astype(jnp.int32)
    e1 = aux[:, 1].astype(jnp.int32)
    phist = phist3.reshape(SC_TILES, LANES)

    dispatch = _make_dispatch(B, D, E, npad, nblk_pad)
    xg, pos, bexp = dispatch(e0, e1, phist, x)

    y = _run_grouped(bexp, xg, We1.astype(jnp.bfloat16), be1,
                     We2.astype(jnp.bfloat16), be2, nblk)

    gather = _make_gather(B, C, npad)
    y0, y1 = gather(y, pos)

    out = _run_combine(y0, y1, aux)

    lb_loss = jnp.asarray(0.0, jnp.float32)
    return (out, router_p, lb_loss)


# speedup vs baseline: 1.1390x; 1.1362x over previous
import functools

import jax
import jax.numpy as jnp
from jax import lax
from jax.experimental import pallas as pl
from jax.experimental.pallas import tpu as pltpu
from jax.experimental.pallas import tpu_sc as plsc

_INTERPRET = False

LANES = 128
NEG = -1e30
BLK = 256
BLK_SHIFT = BLK.bit_length() - 1
SC_TILES = 32
SL = 16


def _dot(a, b):
    return jnp.dot(a.astype(jnp.bfloat16), b.astype(jnp.bfloat16),
                   preferred_element_type=jnp.float32)


def _router_body(x_ref, w1_ref, b1_ref, w2_ref, b2_ref, w3_ref, b3_ref,
                 p_ref, aux_ref, ph_ref):
    xb = x_ref[...]
    h1 = jax.nn.relu(_dot(xb, w1_ref[...]) + b1_ref[...])
    h2 = jax.nn.relu(_dot(h1, w2_ref[...]) + b2_ref[...])
    logits = _dot(h2, w3_ref[...]) + b3_ref[...]

    m = jnp.max(logits, axis=1, keepdims=True)
    ex = jnp.exp(logits - m)
    s = jnp.sum(ex, axis=1, keepdims=True)
    p = ex / s
    p_ref[...] = p

    idx = jax.lax.broadcasted_iota(jnp.int32, p.shape, 1)
    m1 = jnp.max(p, axis=1, keepdims=True)
    e0 = jnp.min(jnp.where(p == m1, idx, LANES), axis=1, keepdims=True)
    pm = jnp.where(idx == e0, -1.0, p)
    m2 = jnp.max(pm, axis=1, keepdims=True)
    e1 = jnp.min(jnp.where(pm == m2, idx, LANES), axis=1, keepdims=True)

    e0f = e0.astype(jnp.float32)
    e1f = e1.astype(jnp.float32)
    aux = jnp.where(idx == 0, e0f,
          jnp.where(idx == 1, e1f,
          jnp.where(idx == 2, m1,
          jnp.where(idx == 3, m2, 0.0))))
    aux_ref[...] = aux

    ind = (jnp.where(idx == e0, 1.0, 0.0) + jnp.where(idx == e1, 1.0, 0.0))
    tb = ind.shape[0]
    nrow = tb // 128
    ph_ref[...] = jnp.sum(ind.reshape(1, nrow, 128, LANES), axis=2)


def _run_router(x, Wr1, br1, Wr2, br2, Wr3, br3):
    B, D = x.shape
    HID = Wr1.shape[1]
    H2 = Wr2.shape[1]
    E = Wr3.shape[1]
    TB = min(512, B)
    nb = B // TB
    nrow = TB // 128

    w3p = jnp.zeros((H2, LANES), Wr3.dtype).at[:, :E].set(Wr3)
    b3p = jnp.full((1, LANES), NEG, br3.dtype).at[0, :E].set(br3)

    out_shapes = (
        jax.ShapeDtypeStruct((B, LANES), jnp.float32),
        jax.ShapeDtypeStruct((B, LANES), jnp.float32),
        jax.ShapeDtypeStruct((nb, nrow, LANES), jnp.float32),
    )
    return pl.pallas_call(
        _router_body,
        grid=(nb,),
        in_specs=[
            pl.BlockSpec((TB, D), lambda i: (i, 0)),
            pl.BlockSpec((D, HID), lambda i: (0, 0)),
            pl.BlockSpec((1, HID), lambda i: (0, 0)),
            pl.BlockSpec((HID, H2), lambda i: (0, 0)),
            pl.BlockSpec((1, H2), lambda i: (0, 0)),
            pl.BlockSpec((H2, LANES), lambda i: (0, 0)),
            pl.BlockSpec((1, LANES), lambda i: (0, 0)),
        ],
        out_specs=(
            pl.BlockSpec((TB, LANES), lambda i: (i, 0)),
            pl.BlockSpec((TB, LANES), lambda i: (i, 0)),
            pl.BlockSpec((1, nrow, LANES), lambda i: (i, 0, 0)),
        ),
        out_shape=out_shapes,
        interpret=_INTERPRET,
    )(x, Wr1, br1.reshape(1, HID), Wr2, br2.reshape(1, H2), w3p, b3p)


def _gather16(v, idx):
    dn = lax.GatherDimensionNumbers(offset_dims=(), collapsed_slice_dims=(0,),
                                    start_index_map=(0,))
    return lax.gather(v, idx[:, None], dn, slice_sizes=(1,),
                      mode=lax.GatherScatterMode.PROMISE_IN_BOUNDS)


def _make_dispatch(B, D, E, npad, nblk_pad):
    tpb = B // SC_TILES
    nv = tpb // SL
    rows_c = 32
    nchunk = 2 * tpb // rows_c
    mesh = plsc.VectorSubcoreMesh(core_axis_name="c", subcore_axis_name="s")
    info = plsc.get_sparse_core_info()
    nc = info.num_cores

    @functools.partial(
        pl.kernel, mesh=mesh,
        out_type=(
            jax.ShapeDtypeStruct((npad, D), jnp.float32),
            jax.ShapeDtypeStruct((2 * B,), jnp.int32),
            jax.ShapeDtypeStruct((nblk_pad,), jnp.int32),
        ),
        scratch_types=[
            pltpu.VMEM((tpb,), jnp.int32),
            pltpu.VMEM((tpb,), jnp.int32),
            pltpu.VMEM((SC_TILES, LANES), jnp.float32),
            pltpu.VMEM((2 * tpb,), jnp.int32),
            pltpu.VMEM((nchunk, rows_c), jnp.int32),
            pltpu.VMEM((nblk_pad,), jnp.int32),
            pltpu.VMEM((rows_c, D), jnp.float32),
            pltpu.VMEM((rows_c, D), jnp.float32),
            pltpu.SemaphoreType.DMA,
            pltpu.SemaphoreType.DMA,
        ],
    )
    def dispatch(e0_hbm, e1_hbm, ph_hbm, x_hbm,
                 xg_hbm, pos_hbm, bexp_hbm,
                 e0_v, e1_v, ph_v, posbuf, pos3, bexp_v,
                 rb0, rb1, sem0, sem1):
        wid = lax.axis_index("s") * nc + lax.axis_index("c")
        base = wid * tpb

        pltpu.sync_copy(e0_hbm.at[pl.ds(base, tpb)], e0_v)
        pltpu.sync_copy(e1_hbm.at[pl.ds(base, tpb)], e1_v)
        pltpu.sync_copy(ph_hbm, ph_v)

        lane = jnp.arange(SL, dtype=jnp.int32)
        total = jnp.zeros((SL,), jnp.int32)
        before = jnp.zeros((SL,), jnp.int32)
        for w in range(SC_TILES):
            v = ph_v[w, pl.ds(0, SL)].astype(jnp.int32)
            total = total + v
            before = before + jnp.where(w < wid, v, 0)

        padded = (total + (BLK - 1)) & (-BLK)
        s = padded
        for k in (1, 2, 4, 8):
            sh = _gather16(s, jnp.maximum(lane - k, 0))
            s = s + jnp.where(lane >= k, sh, 0)
        basev = s - padded
        ctr = basev + before

        for half, eref in ((0, e0_v), (1, e1_v)):
            for i in range(nv):
                veid = eref[pl.ds(i * SL, SL)]
                g = _gather16(ctr, veid)
                rank = jnp.zeros((SL,), jnp.int32)
                for k in range(1, SL):
                    sh = _gather16(veid, jnp.maximum(lane - k, 0))
                    hit = (sh == veid) & (lane >= k)
                    rank = rank + jnp.where(hit, 1, 0)
                upd = jnp.zeros((SL,), jnp.int32)
                for j in range(SL):
                    vj = _gather16(veid, jnp.full((SL,), j, jnp.int32))
                    upd = upd + jnp.where(lane == vj, 1, 0)
                ctr = ctr + upd
                pos = g + rank
                o = half * tpb + i * SL
                posbuf[pl.ds(o, SL)] = pos
                pos3[o // rows_c, pl.ds(o % rows_c, SL)] = pos

        pltpu.sync_copy(posbuf.at[pl.ds(0, tpb)],
                        pos_hbm.at[pl.ds(base, tpb)])
        pltpu.sync_copy(posbuf.at[pl.ds(tpb, tpb)],
                        pos_hbm.at[pl.ds(B + base, tpb)])

        bufs = (rb0, rb1)
        sems = (sem0, sem1)
        pending = [None, None]
        cpt = tpb // rows_c
        for c in range(nchunk):
            b = c % 2
            if pending[b] is not None:
                pending[b].wait()
            tok0 = base + (c % cpt) * rows_c
            pltpu.sync_copy(x_hbm.at[pl.ds(tok0, rows_c)], bufs[b])
            cp = pltpu.make_async_copy(bufs[b], xg_hbm.at[pos3.at[c]],
                                       sems[b])
            cp.start()
            pending[b] = cp
        for b in range(2):
            if pending[b] is not None:
                pending[b].wait()

        @pl.when(wid == 0)
        def _():
            bstarts = jnp.right_shift(basev, BLK_SHIFT)
            for vi in range(nblk_pad // SL):
                jv = jnp.arange(SL, dtype=jnp.int32) + vi * SL
                acc = jnp.full((SL,), -1, jnp.int32)
                for e in range(E):
                    bse = _gather16(bstarts, jnp.full((SL,), e, jnp.int32))
                    acc = acc + jnp.where(jv >= bse, 1, 0)
                bexp_v[pl.ds(vi * SL, SL)] = acc
            pltpu.sync_copy(bexp_v, bexp_hbm)

    return dispatch


def _grouped_body(bexp_ref, xg_ref, w1_ref, b1_ref, w2_ref, b2_ref, y_ref):
    xb = xg_ref[...]
    h = jax.nn.relu(_dot(xb, w1_ref[0]) + b1_ref[0])
    y_ref[...] = _dot(h, w2_ref[0]) + b2_ref[0]


def _run_grouped(bexp, xg, We1, be1, We2, be2, nblk):
    E, D, DFF = We1.shape
    C = We2.shape[2]
    npad = xg.shape[0]

    grid_spec = pltpu.PrefetchScalarGridSpec(
        num_scalar_prefetch=1,
        grid=(nblk,),
        in_specs=[
            pl.BlockSpec((BLK, D), lambda b, be: (b, 0)),
            pl.BlockSpec((1, D, DFF), lambda b, be: (be[b], 0, 0)),
            pl.BlockSpec((1, 1, DFF), lambda b, be: (be[b], 0, 0)),
            pl.BlockSpec((1, DFF, C), lambda b, be: (be[b], 0, 0)),
            pl.BlockSpec((1, 1, C), lambda b, be: (be[b], 0, 0)),
        ],
        out_specs=pl.BlockSpec((BLK, C), lambda b, be: (b, 0)),
    )
    return pl.pallas_call(
        _grouped_body,
        grid_spec=grid_spec,
        out_shape=jax.ShapeDtypeStruct((npad, C), jnp.float32),
        interpret=_INTERPRET,
    )(bexp, xg, We1, be1.reshape(E, 1, DFF), We2, be2.reshape(E, 1, C))


def _make_gather(B, C, npad):
    tpb = B // SC_TILES
    rows_c = 32
    cpt = tpb // rows_c
    mesh = plsc.VectorSubcoreMesh(core_axis_name="c", subcore_axis_name="s")
    info = plsc.get_sparse_core_info()
    nc = info.num_cores

    @functools.partial(
        pl.kernel, mesh=mesh,
        out_type=(
            jax.ShapeDtypeStruct((B, C), jnp.float32),
            jax.ShapeDtypeStruct((B, C), jnp.float32),
        ),
        scratch_types=[
            pltpu.VMEM((tpb,), jnp.int32),
            pltpu.VMEM((tpb,), jnp.int32),
            pltpu.VMEM((rows_c, C), jnp.float32),
            pltpu.VMEM((rows_c, C), jnp.float32),
            pltpu.SemaphoreType.DMA,
            pltpu.SemaphoreType.DMA,
            pltpu.SemaphoreType.DMA,
        ],
    )
    def gather(y_hbm, pos_hbm, y0_hbm, y1_hbm,
               p0_v, p1_v, rb0, rb1, gsem, wsem0, wsem1):
        wid = lax.axis_index("s") * nc + lax.axis_index("c")
        base = wid * tpb

        pltpu.sync_copy(pos_hbm.at[pl.ds(base, tpb)], p0_v)
        pltpu.sync_copy(pos_hbm.at[pl.ds(B + base, tpb)], p1_v)

        bufs = (rb0, rb1)
        wsems = (wsem0, wsem1)
        pending = [None, None]
        for half, pv, yout in ((0, p0_v, y0_hbm), (1, p1_v, y1_hbm)):
            for c in range(cpt):
                b = (half * cpt + c) % 2
                if pending[b] is not None:
                    pending[b].wait()
                gcp = pltpu.make_async_copy(
                    y_hbm.at[pv.at[pl.ds(c * rows_c, rows_c)]],
                    bufs[b], gsem)
                gcp.start()
                gcp.wait()
                wp = pltpu.make_async_copy(
                    bufs[b], yout.at[pl.ds(base + c * rows_c, rows_c)],
                    wsems[b])
                wp.start()
                pending[b] = wp
        for b in range(2):
            if pending[b] is not None:
                pending[b].wait()

    return gather


def _combine_body(y0_ref, y1_ref, aux_ref, out_ref):
    w0 = aux_ref[:, 2:3]
    w1 = aux_ref[:, 3:4]
    out_ref[...] = 0.5 * (w0 * y0_ref[...] + w1 * y1_ref[...])


def _run_combine(y0, y1, aux):
    B, C = y0.shape
    TB = min(512, B)
    return pl.pallas_call(
        _combine_body,
        grid=(B // TB,),
        in_specs=[
            pl.BlockSpec((TB, C), lambda i: (i, 0)),
            pl.BlockSpec((TB, C), lambda i: (i, 0)),
            pl.BlockSpec((TB, LANES), lambda i: (i, 0)),
        ],
        out_specs=pl.BlockSpec((TB, C), lambda i: (i, 0)),
        out_shape=jax.ShapeDtypeStruct((B, C), jnp.float32),
        interpret=_INTERPRET,
    )(y0, y1, aux)


def kernel(x, Wr1, br1, Wr2, br2, Wr3, br3, We1, be1, We2, be2):
    B, D = x.shape
    E, _, DFF = We1.shape
    C = We2.shape[2]
    nblk = 2 * B // BLK + E
    nblk_pad = ((nblk + SL - 1) // SL) * SL
    npad = nblk * BLK

    x_bf = x.astype(jnp.bfloat16)
    router_p_pad, aux, phist3 = _run_router(x_bf, Wr1, br1, Wr2, br2, Wr3, br3)
    router_p = router_p_pad[:, :E]

    e0 = aux[:, 0].astype(jnp.int32)
    e1 = aux[:, 1].astype(jnp.int32)
    phist = phist3.reshape(SC_TILES, LANES)

    dispatch = _make_dispatch(B, D, E, npad, nblk_pad)
    xg, pos, bexp = dispatch(e0, e1, phist, x)

    y = _run_grouped(bexp, xg, We1, be1, We2, be2, nblk)

    gather = _make_gather(B, C, npad)
    y0, y1 = gather(y, pos)

    out = _run_combine(y0, y1, aux)

    lb_loss = jnp.asarray(0.0, jnp.float32)
    return (out, router_p, lb_loss)


# speedup vs baseline: 1.1682x; 1.0256x over previous
import functools

import jax
import jax.numpy as jnp
from jax import lax
from jax.experimental import pallas as pl
from jax.experimental.pallas import tpu as pltpu
from jax.experimental.pallas import tpu_sc as plsc

_INTERPRET = False

LANES = 128
NEG = -1e30
BLK = 256
BLK_SHIFT = BLK.bit_length() - 1
SC_TILES = 32
SL = 16


def _dot(a, b):
    return jnp.dot(a.astype(jnp.bfloat16), b.astype(jnp.bfloat16),
                   preferred_element_type=jnp.float32)


def _router_body(x_ref, w1_ref, b1_ref, w2_ref, b2_ref, w3_ref, b3_ref,
                 p_ref, aux_ref, ph_ref):
    xb = x_ref[...]
    h1 = jax.nn.relu(_dot(xb, w1_ref[...]) + b1_ref[...])
    h2 = jax.nn.relu(_dot(h1, w2_ref[...]) + b2_ref[...])
    logits = _dot(h2, w3_ref[...]) + b3_ref[...]

    m = jnp.max(logits, axis=1, keepdims=True)
    ex = jnp.exp(logits - m)
    s = jnp.sum(ex, axis=1, keepdims=True)
    p = ex / s
    p_ref[...] = p

    idx = jax.lax.broadcasted_iota(jnp.int32, p.shape, 1)
    m1 = jnp.max(p, axis=1, keepdims=True)
    e0 = jnp.min(jnp.where(p == m1, idx, LANES), axis=1, keepdims=True)
    pm = jnp.where(idx == e0, -1.0, p)
    m2 = jnp.max(pm, axis=1, keepdims=True)
    e1 = jnp.min(jnp.where(pm == m2, idx, LANES), axis=1, keepdims=True)

    e0f = e0.astype(jnp.float32)
    e1f = e1.astype(jnp.float32)
    aux = jnp.where(idx == 0, e0f,
          jnp.where(idx == 1, e1f,
          jnp.where(idx == 2, m1,
          jnp.where(idx == 3, m2, 0.0))))
    aux_ref[...] = aux

    ind = (jnp.where(idx == e0, 1.0, 0.0) + jnp.where(idx == e1, 1.0, 0.0))
    tb = ind.shape[0]
    nrow = tb // 128
    ph_ref[...] = jnp.sum(ind.reshape(1, nrow, 128, LANES), axis=2)


def _run_router(x, Wr1, br1, Wr2, br2, Wr3, br3):
    B, D = x.shape
    HID = Wr1.shape[1]
    H2 = Wr2.shape[1]
    E = Wr3.shape[1]
    TB = min(512, B)
    nb = B // TB
    nrow = TB // 128

    w3p = jnp.zeros((H2, LANES), Wr3.dtype).at[:, :E].set(Wr3)
    b3p = jnp.full((1, LANES), NEG, br3.dtype).at[0, :E].set(br3)

    out_shapes = (
        jax.ShapeDtypeStruct((B, LANES), jnp.float32),
        jax.ShapeDtypeStruct((B, LANES), jnp.float32),
        jax.ShapeDtypeStruct((nb, nrow, LANES), jnp.float32),
    )
    return pl.pallas_call(
        _router_body,
        grid=(nb,),
        in_specs=[
            pl.BlockSpec((TB, D), lambda i: (i, 0)),
            pl.BlockSpec((D, HID), lambda i: (0, 0)),
            pl.BlockSpec((1, HID), lambda i: (0, 0)),
            pl.BlockSpec((HID, H2), lambda i: (0, 0)),
            pl.BlockSpec((1, H2), lambda i: (0, 0)),
            pl.BlockSpec((H2, LANES), lambda i: (0, 0)),
            pl.BlockSpec((1, LANES), lambda i: (0, 0)),
        ],
        out_specs=(
            pl.BlockSpec((TB, LANES), lambda i: (i, 0)),
            pl.BlockSpec((TB, LANES), lambda i: (i, 0)),
            pl.BlockSpec((1, nrow, LANES), lambda i: (i, 0, 0)),
        ),
        out_shape=out_shapes,
        interpret=_INTERPRET,
    )(x, Wr1, br1.reshape(1, HID), Wr2, br2.reshape(1, H2), w3p, b3p)


def _gather16(v, idx):
    dn = lax.GatherDimensionNumbers(offset_dims=(), collapsed_slice_dims=(0,),
                                    start_index_map=(0,))
    return lax.gather(v, idx[:, None], dn, slice_sizes=(1,),
                      mode=lax.GatherScatterMode.PROMISE_IN_BOUNDS)


def _make_dispatch(B, D, E, npad, nblk_pad):
    tpb = B // SC_TILES
    nv = tpb // SL
    rows_c = 32
    nchunk = 2 * tpb // rows_c
    mesh = plsc.VectorSubcoreMesh(core_axis_name="c", subcore_axis_name="s")
    info = plsc.get_sparse_core_info()
    nc = info.num_cores

    @functools.partial(
        pl.kernel, mesh=mesh,
        out_type=(
            jax.ShapeDtypeStruct((npad, D), jnp.float32),
            jax.ShapeDtypeStruct((2 * B,), jnp.int32),
            jax.ShapeDtypeStruct((nblk_pad,), jnp.int32),
        ),
        scratch_types=[
            pltpu.VMEM((tpb,), jnp.int32),
            pltpu.VMEM((tpb,), jnp.int32),
            pltpu.VMEM((SC_TILES, LANES), jnp.float32),
            pltpu.VMEM((2 * tpb,), jnp.int32),
            pltpu.VMEM((nchunk, rows_c), jnp.int32),
            pltpu.VMEM((nblk_pad,), jnp.int32),
            pltpu.VMEM((rows_c, D), jnp.float32),
            pltpu.VMEM((rows_c, D), jnp.float32),
            pltpu.SemaphoreType.DMA,
            pltpu.SemaphoreType.DMA,
        ],
    )
    def dispatch(e0_hbm, e1_hbm, ph_hbm, x_hbm,
                 xg_hbm, pos_hbm, bexp_hbm,
                 e0_v, e1_v, ph_v, posbuf, pos3, bexp_v,
                 rb0, rb1, sem0, sem1):
        wid = lax.axis_index("s") * nc + lax.axis_index("c")
        base = wid * tpb

        pltpu.sync_copy(e0_hbm.at[pl.ds(base, tpb)], e0_v)
        pltpu.sync_copy(e1_hbm.at[pl.ds(base, tpb)], e1_v)
        pltpu.sync_copy(ph_hbm, ph_v)

        lane = jnp.arange(SL, dtype=jnp.int32)
        total = jnp.zeros((SL,), jnp.int32)
        before = jnp.zeros((SL,), jnp.int32)
        for w in range(SC_TILES):
            v = ph_v[w, pl.ds(0, SL)].astype(jnp.int32)
            total = total + v
            before = before + jnp.where(w < wid, v, 0)

        padded = (total + (BLK - 1)) & (-BLK)
        s = padded
        for k in (1, 2, 4, 8):
            sh = _gather16(s, jnp.maximum(lane - k, 0))
            s = s + jnp.where(lane >= k, sh, 0)
        basev = s - padded
        ctr = basev + before

        for half, eref in ((0, e0_v), (1, e1_v)):
            for i in range(nv):
                veid = eref[pl.ds(i * SL, SL)]
                g = _gather16(ctr, veid)
                rank = jnp.zeros((SL,), jnp.int32)
                for k in range(1, SL):
                    sh = _gather16(veid, jnp.maximum(lane - k, 0))
                    hit = (sh == veid) & (lane >= k)
                    rank = rank + jnp.where(hit, 1, 0)
                upd = jnp.zeros((SL,), jnp.int32)
                for j in range(SL):
                    vj = _gather16(veid, jnp.full((SL,), j, jnp.int32))
                    upd = upd + jnp.where(lane == vj, 1, 0)
                ctr = ctr + upd
                pos = g + rank
                o = half * tpb + i * SL
                posbuf[pl.ds(o, SL)] = pos
                pos3[o // rows_c, pl.ds(o % rows_c, SL)] = pos

        pltpu.sync_copy(posbuf.at[pl.ds(0, tpb)],
                        pos_hbm.at[pl.ds(base, tpb)])
        pltpu.sync_copy(posbuf.at[pl.ds(tpb, tpb)],
                        pos_hbm.at[pl.ds(B + base, tpb)])

        bufs = (rb0, rb1)
        sems = (sem0, sem1)
        pending = [None, None]
        cpt = tpb // rows_c
        for c in range(nchunk):
            b = c % 2
            if pending[b] is not None:
                pending[b].wait()
            tok0 = base + (c % cpt) * rows_c
            pltpu.sync_copy(x_hbm.at[pl.ds(tok0, rows_c)], bufs[b])
            cp = pltpu.make_async_copy(bufs[b], xg_hbm.at[pos3.at[c]],
                                       sems[b])
            cp.start()
            pending[b] = cp
        for b in range(2):
            if pending[b] is not None:
                pending[b].wait()

        @pl.when(wid == 0)
        def _():
            bstarts = jnp.right_shift(basev, BLK_SHIFT)
            for vi in range(nblk_pad // SL):
                jv = jnp.arange(SL, dtype=jnp.int32) + vi * SL
                acc = jnp.full((SL,), -1, jnp.int32)
                for e in range(E):
                    bse = _gather16(bstarts, jnp.full((SL,), e, jnp.int32))
                    acc = acc + jnp.where(jv >= bse, 1, 0)
                bexp_v[pl.ds(vi * SL, SL)] = acc
            pltpu.sync_copy(bexp_v, bexp_hbm)

    return dispatch


def _grouped_body(bexp_ref, xg_ref, w1_ref, b1_ref, w2_ref, b2_ref, y_ref):
    xb = xg_ref[...]
    h = jax.nn.relu(_dot(xb, w1_ref[0]) + b1_ref[0])
    y_ref[...] = _dot(h, w2_ref[0]) + b2_ref[0]


def _run_grouped(bexp, xg, We1, be1, We2, be2, nblk):
    E, D, DFF = We1.shape
    C = We2.shape[2]
    npad = xg.shape[0]

    grid_spec = pltpu.PrefetchScalarGridSpec(
        num_scalar_prefetch=1,
        grid=(nblk,),
        in_specs=[
            pl.BlockSpec((BLK, D), lambda b, be: (b, 0)),
            pl.BlockSpec((1, D, DFF), lambda b, be: (be[b], 0, 0)),
            pl.BlockSpec((1, 1, DFF), lambda b, be: (be[b], 0, 0)),
            pl.BlockSpec((1, DFF, C), lambda b, be: (be[b], 0, 0)),
            pl.BlockSpec((1, 1, C), lambda b, be: (be[b], 0, 0)),
        ],
        out_specs=pl.BlockSpec((BLK, C), lambda b, be: (b, 0)),
    )
    return pl.pallas_call(
        _grouped_body,
        grid_spec=grid_spec,
        out_shape=jax.ShapeDtypeStruct((npad, C), jnp.float32),
        interpret=_INTERPRET,
    )(bexp, xg, We1, be1.reshape(E, 1, DFF), We2, be2.reshape(E, 1, C))


def _make_combine(B, C, npad):
    tpb = B // SC_TILES
    rows_c = 16
    cpt = tpb // rows_c
    mesh = plsc.VectorSubcoreMesh(core_axis_name="c", subcore_axis_name="s")
    info = plsc.get_sparse_core_info()
    nc = info.num_cores

    @functools.partial(
        pl.kernel, mesh=mesh,
        out_type=jax.ShapeDtypeStruct((B, C), jnp.float32),
        scratch_types=[
            pltpu.VMEM((tpb,), jnp.int32),
            pltpu.VMEM((tpb,), jnp.int32),
            pltpu.VMEM((tpb,), jnp.float32),
            pltpu.VMEM((tpb,), jnp.float32),
            pltpu.VMEM((2, rows_c, C), jnp.float32),
            pltpu.VMEM((2, rows_c, C), jnp.float32),
            pltpu.SemaphoreType.DMA,
            pltpu.SemaphoreType.DMA,
            pltpu.SemaphoreType.DMA,
            pltpu.SemaphoreType.DMA,
            pltpu.SemaphoreType.DMA,
            pltpu.SemaphoreType.DMA,
        ],
    )
    def combine(y_hbm, pos_hbm, w0_hbm, w1_hbm, out_hbm,
                p0_v, p1_v, w0_v, w1_v, ra, rb,
                g0s0, g0s1, g1s0, g1s1, ws0, ws1):
        wid = lax.axis_index("s") * nc + lax.axis_index("c")
        base = wid * tpb

        pltpu.sync_copy(pos_hbm.at[pl.ds(base, tpb)], p0_v)
        pltpu.sync_copy(pos_hbm.at[pl.ds(B + base, tpb)], p1_v)
        pltpu.sync_copy(w0_hbm.at[pl.ds(base, tpb)], w0_v)
        pltpu.sync_copy(w1_hbm.at[pl.ds(base, tpb)], w1_v)

        g0sems = (g0s0, g0s1)
        g1sems = (g1s0, g1s1)
        wsems = (ws0, ws1)
        pend_g = [None, None]
        pend_w = [None, None]

        def start(c):
            b = c & 1
            if pend_w[b] is not None:
                pend_w[b].wait()
                pend_w[b] = None
            g0 = pltpu.make_async_copy(
                y_hbm.at[p0_v.at[pl.ds(c * rows_c, rows_c)]],
                ra.at[b], g0sems[b])
            g1 = pltpu.make_async_copy(
                y_hbm.at[p1_v.at[pl.ds(c * rows_c, rows_c)]],
                rb.at[b], g1sems[b])
            g0.start()
            g1.start()
            pend_g[b] = (g0, g1)

        start(0)
        for c in range(cpt):
            b = c & 1
            if c + 1 < cpt:
                start(c + 1)
            for p in pend_g[b]:
                p.wait()
            w0c = w0_v[pl.ds(c * rows_c, SL)]
            w1c = w1_v[pl.ds(c * rows_c, SL)]

            @plsc.parallel_loop(0, rows_c)
            def _rows(r):
                wv0 = _gather16(w0c, jnp.full((SL,), r, jnp.int32))
                wv1 = _gather16(w1c, jnp.full((SL,), r, jnp.int32))

                @plsc.parallel_loop(0, C // SL)
                def _cols(g):
                    sl = pl.ds(g * SL, SL)
                    ra[b, r, sl] = wv0 * ra[b, r, sl] + wv1 * rb[b, r, sl]
            wp = pltpu.make_async_copy(
                ra.at[b], out_hbm.at[pl.ds(base + c * rows_c, rows_c)],
                wsems[b])
            wp.start()
            pend_w[b] = wp
        for b in range(2):
            if pend_w[b] is not None:
                pend_w[b].wait()

    return combine


def kernel(x, Wr1, br1, Wr2, br2, Wr3, br3, We1, be1, We2, be2):
    B, D = x.shape
    E, _, DFF = We1.shape
    C = We2.shape[2]
    nblk = 2 * B // BLK + E
    nblk_pad = ((nblk + SL - 1) // SL) * SL
    npad = nblk * BLK

    x_bf = x.astype(jnp.bfloat16)
    router_p_pad, aux, phist3 = _run_router(x_bf, Wr1, br1, Wr2, br2, Wr3, br3)
    router_p = router_p_pad[:, :E]

    e0 = aux[:, 0].astype(jnp.int32)
    e1 = aux[:, 1].astype(jnp.int32)
    phist = phist3.reshape(SC_TILES, LANES)

    dispatch = _make_dispatch(B, D, E, npad, nblk_pad)
    xg, pos, bexp = dispatch(e0, e1, phist, x)

    y = _run_grouped(bexp, xg, We1, be1, We2, be2, nblk)

    w0 = 0.5 * aux[:, 2]
    w1 = 0.5 * aux[:, 3]
    combine = _make_combine(B, C, npad)
    out = combine(y, pos, w0, w1)

    lb_loss = jnp.asarray(0.0, jnp.float32)
    return (out, router_p, lb_loss)


# speedup vs baseline: 1.2449x; 1.0657x over previous
import functools

import jax
import jax.numpy as jnp
from jax import lax
from jax.experimental import pallas as pl
from jax.experimental.pallas import tpu as pltpu
from jax.experimental.pallas import tpu_sc as plsc

_INTERPRET = False

LANES = 128
NEG = -1e30
BLK = 256
BLK_SHIFT = BLK.bit_length() - 1
SC_TILES = 32
SL = 16


def _dot(a, b):
    return jnp.dot(a.astype(jnp.bfloat16), b.astype(jnp.bfloat16),
                   preferred_element_type=jnp.float32)


def _router_body(x_ref, w1_ref, b1_ref, w2_ref, b2_ref, w3_ref, b3_ref,
                 p_ref, aux_ref, ph_ref):
    xb = x_ref[...]
    h1 = jax.nn.relu(_dot(xb, w1_ref[...]) + b1_ref[...])
    h2 = jax.nn.relu(_dot(h1, w2_ref[...]) + b2_ref[...])
    logits = _dot(h2, w3_ref[...]) + b3_ref[...]

    m = jnp.max(logits, axis=1, keepdims=True)
    ex = jnp.exp(logits - m)
    s = jnp.sum(ex, axis=1, keepdims=True)
    p = ex / s
    p_ref[...] = p

    idx = jax.lax.broadcasted_iota(jnp.int32, p.shape, 1)
    m1 = jnp.max(p, axis=1, keepdims=True)
    e0 = jnp.min(jnp.where(p == m1, idx, LANES), axis=1, keepdims=True)
    pm = jnp.where(idx == e0, -1.0, p)
    m2 = jnp.max(pm, axis=1, keepdims=True)
    e1 = jnp.min(jnp.where(pm == m2, idx, LANES), axis=1, keepdims=True)

    e0f = e0.astype(jnp.float32)
    e1f = e1.astype(jnp.float32)
    aux = jnp.where(idx == 0, e0f,
          jnp.where(idx == 1, e1f,
          jnp.where(idx == 2, m1,
          jnp.where(idx == 3, m2, 0.0))))
    aux_ref[...] = aux

    ind = (jnp.where(idx == e0, 1.0, 0.0) + jnp.where(idx == e1, 1.0, 0.0))
    tb = ind.shape[0]
    nrow = tb // 128
    ph_ref[...] = jnp.sum(ind.reshape(1, nrow, 128, LANES), axis=2)


def _run_router(x, Wr1, br1, Wr2, br2, Wr3, br3):
    B, D = x.shape
    HID = Wr1.shape[1]
    H2 = Wr2.shape[1]
    E = Wr3.shape[1]
    TB = min(512, B)
    nb = B // TB
    nrow = TB // 128

    w3p = jnp.zeros((H2, LANES), Wr3.dtype).at[:, :E].set(Wr3)
    b3p = jnp.full((1, LANES), NEG, br3.dtype).at[0, :E].set(br3)

    out_shapes = (
        jax.ShapeDtypeStruct((B, LANES), jnp.float32),
        jax.ShapeDtypeStruct((B, LANES), jnp.float32),
        jax.ShapeDtypeStruct((nb, nrow, LANES), jnp.float32),
    )
    return pl.pallas_call(
        _router_body,
        grid=(nb,),
        in_specs=[
            pl.BlockSpec((TB, D), lambda i: (i, 0)),
            pl.BlockSpec((D, HID), lambda i: (0, 0)),
            pl.BlockSpec((1, HID), lambda i: (0, 0)),
            pl.BlockSpec((HID, H2), lambda i: (0, 0)),
            pl.BlockSpec((1, H2), lambda i: (0, 0)),
            pl.BlockSpec((H2, LANES), lambda i: (0, 0)),
            pl.BlockSpec((1, LANES), lambda i: (0, 0)),
        ],
        out_specs=(
            pl.BlockSpec((TB, LANES), lambda i: (i, 0)),
            pl.BlockSpec((TB, LANES), lambda i: (i, 0)),
            pl.BlockSpec((1, nrow, LANES), lambda i: (i, 0, 0)),
        ),
        out_shape=out_shapes,
        interpret=_INTERPRET,
    )(x, Wr1, br1.reshape(1, HID), Wr2, br2.reshape(1, H2), w3p, b3p)


def _gather16(v, idx):
    dn = lax.GatherDimensionNumbers(offset_dims=(), collapsed_slice_dims=(0,),
                                    start_index_map=(0,))
    return lax.gather(v, idx[:, None], dn, slice_sizes=(1,),
                      mode=lax.GatherScatterMode.PROMISE_IN_BOUNDS)


def _make_dispatch(B, D, E, npad, nblk_pad):
    tpb = B // SC_TILES
    nv = tpb // SL
    rows_c = 32
    nchunk = 2 * tpb // rows_c
    mesh = plsc.VectorSubcoreMesh(core_axis_name="c", subcore_axis_name="s")
    info = plsc.get_sparse_core_info()
    nc = info.num_cores

    @functools.partial(
        pl.kernel, mesh=mesh,
        out_type=(
            jax.ShapeDtypeStruct((npad, D), jnp.float32),
            jax.ShapeDtypeStruct((2 * B,), jnp.int32),
            jax.ShapeDtypeStruct((nblk_pad,), jnp.int32),
        ),
        scratch_types=[
            pltpu.VMEM((tpb,), jnp.int32),
            pltpu.VMEM((tpb,), jnp.int32),
            pltpu.VMEM((SC_TILES, LANES), jnp.float32),
            pltpu.VMEM((2 * tpb,), jnp.int32),
            pltpu.VMEM((nchunk, rows_c), jnp.int32),
            pltpu.VMEM((nblk_pad,), jnp.int32),
            pltpu.VMEM((rows_c, D), jnp.float32),
            pltpu.VMEM((rows_c, D), jnp.float32),
            pltpu.SemaphoreType.DMA,
            pltpu.SemaphoreType.DMA,
        ],
    )
    def dispatch(e0_hbm, e1_hbm, ph_hbm, x_hbm,
                 xg_hbm, pos_hbm, bexp_hbm,
                 e0_v, e1_v, ph_v, posbuf, pos3, bexp_v,
                 rb0, rb1, sem0, sem1):
        wid = lax.axis_index("s") * nc + lax.axis_index("c")
        base = wid * tpb

        pltpu.sync_copy(e0_hbm.at[pl.ds(base, tpb)], e0_v)
        pltpu.sync_copy(e1_hbm.at[pl.ds(base, tpb)], e1_v)
        pltpu.sync_copy(ph_hbm, ph_v)

        lane = jnp.arange(SL, dtype=jnp.int32)
        total = jnp.zeros((SL,), jnp.int32)
        before = jnp.zeros((SL,), jnp.int32)
        for w in range(SC_TILES):
            v = ph_v[w, pl.ds(0, SL)].astype(jnp.int32)
            total = total + v
            before = before + jnp.where(w < wid, v, 0)

        padded = (total + (BLK - 1)) & (-BLK)
        s = padded
        for k in (1, 2, 4, 8):
            sh = _gather16(s, jnp.maximum(lane - k, 0))
            s = s + jnp.where(lane >= k, sh, 0)
        basev = s - padded
        ctr = basev + before

        for half, eref in ((0, e0_v), (1, e1_v)):
            for i in range(nv):
                veid = eref[pl.ds(i * SL, SL)]
                g = _gather16(ctr, veid)
                rank = jnp.zeros((SL,), jnp.int32)
                for k in range(1, SL):
                    sh = _gather16(veid, jnp.maximum(lane - k, 0))
                    hit = (sh == veid) & (lane >= k)
                    rank = rank + jnp.where(hit, 1, 0)
                upd = jnp.zeros((SL,), jnp.int32)
                for j in range(SL):
                    vj = _gather16(veid, jnp.full((SL,), j, jnp.int32))
                    upd = upd + jnp.where(lane == vj, 1, 0)
                ctr = ctr + upd
                pos = g + rank
                o = half * tpb + i * SL
                posbuf[pl.ds(o, SL)] = pos
                pos3[o // rows_c, pl.ds(o % rows_c, SL)] = pos

        pltpu.sync_copy(posbuf.at[pl.ds(0, tpb)],
                        pos_hbm.at[pl.ds(base, tpb)])
        pltpu.sync_copy(posbuf.at[pl.ds(tpb, tpb)],
                        pos_hbm.at[pl.ds(B + base, tpb)])

        bufs = (rb0, rb1)
        sems = (sem0, sem1)
        pending = [None, None]
        cpt = tpb // rows_c
        for c in range(nchunk):
            b = c % 2
            if pending[b] is not None:
                pending[b].wait()
            tok0 = base + (c % cpt) * rows_c
            pltpu.sync_copy(x_hbm.at[pl.ds(tok0, rows_c)], bufs[b])
            cp = pltpu.make_async_copy(bufs[b], xg_hbm.at[pos3.at[c]],
                                       sems[b])
            cp.start()
            pending[b] = cp
        for b in range(2):
            if pending[b] is not None:
                pending[b].wait()

        @pl.when(wid == 0)
        def _():
            bstarts = jnp.right_shift(basev, BLK_SHIFT)
            for vi in range(nblk_pad // SL):
                jv = jnp.arange(SL, dtype=jnp.int32) + vi * SL
                acc = jnp.full((SL,), -1, jnp.int32)
                for e in range(E):
                    bse = _gather16(bstarts, jnp.full((SL,), e, jnp.int32))
                    acc = acc + jnp.where(jv >= bse, 1, 0)
                bexp_v[pl.ds(vi * SL, SL)] = acc
            pltpu.sync_copy(bexp_v, bexp_hbm)

    return dispatch


def _grouped_body(bexp_ref, xg_ref, w1_ref, b1_ref, w2_ref, b2_ref, y_ref):
    xb = xg_ref[...]
    h = jax.nn.relu(_dot(xb, w1_ref[0]) + b1_ref[0])
    y_ref[...] = _dot(h, w2_ref[0]) + b2_ref[0]


def _run_grouped(bexp, xg, We1, be1, We2, be2, nblk):
    E, D, DFF = We1.shape
    C = We2.shape[2]
    npad = xg.shape[0]

    grid_spec = pltpu.PrefetchScalarGridSpec(
        num_scalar_prefetch=1,
        grid=(nblk,),
        in_specs=[
            pl.BlockSpec((BLK, D), lambda b, be: (b, 0)),
            pl.BlockSpec((1, D, DFF), lambda b, be: (be[b], 0, 0)),
            pl.BlockSpec((1, 1, DFF), lambda b, be: (be[b], 0, 0)),
            pl.BlockSpec((1, DFF, C), lambda b, be: (be[b], 0, 0)),
            pl.BlockSpec((1, 1, C), lambda b, be: (be[b], 0, 0)),
        ],
        out_specs=pl.BlockSpec((BLK, C), lambda b, be: (b, 0)),
    )
    return pl.pallas_call(
        _grouped_body,
        grid_spec=grid_spec,
        out_shape=jax.ShapeDtypeStruct((npad, C), jnp.float32),
        interpret=_INTERPRET,
    )(bexp, xg, We1, be1.reshape(E, 1, DFF), We2, be2.reshape(E, 1, C))


def _make_combine(B, C, npad):
    tpb = B // SC_TILES
    rows_c = 16
    cpt = tpb // rows_c
    mesh = plsc.VectorSubcoreMesh(core_axis_name="c", subcore_axis_name="s")
    info = plsc.get_sparse_core_info()
    nc = info.num_cores

    @functools.partial(
        pl.kernel, mesh=mesh,
        out_type=jax.ShapeDtypeStruct((B, C), jnp.float32),
        scratch_types=[
            pltpu.VMEM((tpb,), jnp.int32),
            pltpu.VMEM((tpb,), jnp.int32),
            pltpu.VMEM((tpb,), jnp.float32),
            pltpu.VMEM((tpb,), jnp.float32),
            pltpu.VMEM((2, rows_c, C), jnp.float32),
            pltpu.VMEM((2, rows_c, C), jnp.float32),
            pltpu.SemaphoreType.DMA,
            pltpu.SemaphoreType.DMA,
            pltpu.SemaphoreType.DMA,
            pltpu.SemaphoreType.DMA,
            pltpu.SemaphoreType.DMA,
            pltpu.SemaphoreType.DMA,
        ],
    )
    def combine(y_hbm, pos_hbm, w0_hbm, w1_hbm, out_hbm,
                p0_v, p1_v, w0_v, w1_v, ra, rb,
                g0s0, g0s1, g1s0, g1s1, ws0, ws1):
        wid = lax.axis_index("s") * nc + lax.axis_index("c")
        base = wid * tpb

        pltpu.sync_copy(pos_hbm.at[pl.ds(base, tpb)], p0_v)
        pltpu.sync_copy(pos_hbm.at[pl.ds(B + base, tpb)], p1_v)
        pltpu.sync_copy(w0_hbm.at[pl.ds(base, tpb)], w0_v)
        pltpu.sync_copy(w1_hbm.at[pl.ds(base, tpb)], w1_v)

        g0sems = (g0s0, g0s1)
        g1sems = (g1s0, g1s1)
        wsems = (ws0, ws1)
        pend_g = [None, None]
        pend_w = [None, None]

        def start(c):
            b = c & 1
            if pend_w[b] is not None:
                pend_w[b].wait()
                pend_w[b] = None
            g0 = pltpu.make_async_copy(
                y_hbm.at[p0_v.at[pl.ds(c * rows_c, rows_c)]],
                ra.at[b], g0sems[b])
            g1 = pltpu.make_async_copy(
                y_hbm.at[p1_v.at[pl.ds(c * rows_c, rows_c)]],
                rb.at[b], g1sems[b])
            g0.start()
            g1.start()
            pend_g[b] = (g0, g1)

        start(0)
        for c in range(cpt):
            b = c & 1
            if c + 1 < cpt:
                start(c + 1)
            for p in pend_g[b]:
                p.wait()
            w0c = w0_v[pl.ds(c * rows_c, SL)]
            w1c = w1_v[pl.ds(c * rows_c, SL)]

            @plsc.parallel_loop(0, rows_c)
            def _rows(r):
                wv0 = _gather16(w0c, jnp.full((SL,), r, jnp.int32))
                wv1 = _gather16(w1c, jnp.full((SL,), r, jnp.int32))

                @plsc.parallel_loop(0, C // SL, unroll=4)
                def _cols(g):
                    sl = pl.ds(g * SL, SL)
                    ra[b, r, sl] = wv0 * ra[b, r, sl] + wv1 * rb[b, r, sl]
            wp = pltpu.make_async_copy(
                ra.at[b], out_hbm.at[pl.ds(base + c * rows_c, rows_c)],
                wsems[b])
            wp.start()
            pend_w[b] = wp
        for b in range(2):
            if pend_w[b] is not None:
                pend_w[b].wait()

    return combine


def kernel(x, Wr1, br1, Wr2, br2, Wr3, br3, We1, be1, We2, be2):
    B, D = x.shape
    E, _, DFF = We1.shape
    C = We2.shape[2]
    nblk = 2 * B // BLK + E
    nblk_pad = ((nblk + SL - 1) // SL) * SL
    npad = nblk * BLK

    x_bf = x.astype(jnp.bfloat16)
    router_p_pad, aux, phist3 = _run_router(x_bf, Wr1, br1, Wr2, br2, Wr3, br3)
    router_p = router_p_pad[:, :E]

    e0 = aux[:, 0].astype(jnp.int32)
    e1 = aux[:, 1].astype(jnp.int32)
    phist = phist3.reshape(SC_TILES, LANES)

    dispatch = _make_dispatch(B, D, E, npad, nblk_pad)
    xg, pos, bexp = dispatch(e0, e1, phist, x)

    y = _run_grouped(bexp, xg, We1, be1, We2, be2, nblk)

    w0 = 0.5 * aux[:, 2]
    w1 = 0.5 * aux[:, 3]
    combine = _make_combine(B, C, npad)
    out = combine(y, pos, w0, w1)

    lb_loss = jnp.asarray(0.0, jnp.float32)
    return (out, router_p, lb_loss)
